# baseline (speedup 1.0000x reference)
import jax
import jax.numpy as jnp
from jax import lax
from jax.experimental import pallas as pl
from jax.experimental.pallas import tpu as pltpu
from jax.experimental.pallas import tpu_sc as plsc

B, S, D = 4, 4096, 768
DFF = 3072
EPS = 1e-05
K = max(1, int(0.5 * S))
C = K + 128
BC = B * C
PER_TILE = BC // 32
EXP_TILE = B * S // 32

ROUTER_BLK = 1024
TOK_BLK = 256


def _router_body(h_ref, rw_ref, rb_ref, w_ref):
    logits = jnp.dot(h_ref[...], rw_ref[...], preferred_element_type=jnp.float32)
    w_ref[...] = jax.nn.sigmoid(logits + rb_ref[0])


def _slot_body(wrow_ref, wcol_ref, slotg_ref, mask_ref, idx_ref):
    b = pl.program_id(0)
    w_row = wrow_ref[0]
    w_col = wcol_ref[0]
    i_idx = lax.broadcasted_iota(jnp.int32, (1, S), 1)
    cnt = jnp.zeros((1, S), jnp.float32)
    tie = jnp.zeros((1, S), jnp.float32)
    CH = 512
    for c in range(S // CH):
        wc = w_col[c * CH:(c + 1) * CH, :]
        j_idx = lax.broadcasted_iota(jnp.int32, (CH, 1), 0) + (c * CH)
        cnt = cnt + jnp.sum((wc > w_row).astype(jnp.float32), axis=0,
                            keepdims=True)
        eq = (wc == w_row) & (j_idx < i_idx)
        tie = tie + jnp.sum(eq.astype(jnp.float32), axis=0, keepdims=True)
    sel = cnt < float(K)
    slot = cnt + tie
    mask_ref[0] = sel.astype(jnp.float32)
    slotg_ref[0] = (b * C + jnp.where(sel, slot, 0.0)).astype(jnp.int32)

    p_col = lax.broadcasted_iota(jnp.int32, (C, 1), 0).astype(jnp.float32)
    tok_row = (lax.broadcasted_iota(jnp.int32, (1, S), 1).astype(jnp.float32)
               + (b * S).astype(jnp.float32))
    slot_sel = jnp.where(sel, slot, -1.0)
    acc = jnp.zeros((C, 1), jnp.float32)
    for c in range(S // CH):
        sl_c = slot_sel[:, c * CH:(c + 1) * CH]
        tk_c = tok_row[:, c * CH:(c + 1) * CH]
        onehot = (p_col == sl_c).astype(jnp.float32)
        acc = acc + jnp.sum(onehot * tk_c, axis=1, keepdims=True)
    idx_ref[0] = acc.astype(jnp.int32)


def _block_body(x_ref, wv_ref, wo_ref, g1_ref, g2_ref,
                wg_ref, wu_ref, wd_ref, o_ref, w2_ref):
    @pl.when(pl.program_id(0) == 0)
    def _():
        w2 = jnp.dot(wv_ref[...], wo_ref[...],
                     preferred_element_type=jnp.float32)
        w2_ref[...] = w2.astype(jnp.bfloat16)

    x = x_ref[...]
    n1 = x * lax.rsqrt(jnp.mean(x * x, axis=-1, keepdims=True) + EPS)
    n1 = (n1 * g1_ref[...]).astype(jnp.bfloat16)
    attn = jnp.dot(n1, w2_ref[...], preferred_element_type=jnp.float32)
    r = x + attn
    n2 = r * lax.rsqrt(jnp.mean(r * r, axis=-1, keepdims=True) + EPS)
    n2 = (n2 * g2_ref[...]).astype(jnp.bfloat16)
    gg = jnp.dot(n2, wg_ref[...], preferred_element_type=jnp.float32)
    uu = jnp.dot(n2, wu_ref[...], preferred_element_type=jnp.float32)
    h = ((gg * jax.nn.sigmoid(gg)) * uu).astype(jnp.bfloat16)
    f = jnp.dot(h, wd_ref[...], preferred_element_type=jnp.float32)
    o_ref[...] = r + f


def _sc_compact_body(idx_hbm, hid_hbm, xg_hbm, idxv, rows, sem):
    c = lax.axis_index("c")
    s = lax.axis_index("s")
    base = (s * 2 + c) * PER_TILE
    pltpu.sync_copy(idx_hbm.at[pl.ds(base, PER_TILE)], idxv)
    for off, n in ((0, 128), (128, 128), (256, 16)):
        stage = rows.at[pl.ds(0, n)]
        pltpu.async_copy(hid_hbm.at[idxv.at[pl.ds(off, n)]], stage, sem).wait()
        pltpu.sync_copy(stage, xg_hbm.at[pl.ds(base + off, n)])


def _sc_expand_body(slotg_hbm, yg_hbm, zg_hbm, idxv, rows, sem):
    c = lax.axis_index("c")
    s = lax.axis_index("s")
    base = (s * 2 + c) * EXP_TILE
    pltpu.sync_copy(slotg_hbm.at[pl.ds(base, EXP_TILE)], idxv)
    for j in range(4):
        stage = rows.at[pl.ds(0, 128)]
        pltpu.async_copy(yg_hbm.at[idxv.at[pl.ds(j * 128, 128)]],
                         stage, sem).wait()
        pltpu.sync_copy(stage, zg_hbm.at[pl.ds(base + j * 128, 128)])


def _blend_body(x_ref, z_ref, m_ref, o_ref):
    x = x_ref[...]
    m = m_ref[...]
    o_ref[...] = x + m * (z_ref[...] - x)


def kernel(hidden_states, router_w, router_b, wq, wk, wv, wo, g1, g2, wg, wu, wd):
    del wq, wk
    hid = hidden_states.reshape(B * S, D)

    weights = pl.pallas_call(
        _router_body,
        out_shape=jax.ShapeDtypeStruct((B * S, 1), jnp.float32),
        grid=(B * S // ROUTER_BLK,),
        in_specs=[
            pl.BlockSpec((ROUTER_BLK, D), lambda i: (i, 0)),
            pl.BlockSpec((D, 1), lambda i: (0, 0)),
            pl.BlockSpec(memory_space=pltpu.SMEM),
        ],
        out_specs=pl.BlockSpec((ROUTER_BLK, 1), lambda i: (i, 0)),
    )(hid, router_w, router_b)

    w_row3 = weights.reshape(B, 1, S)
    w_col3 = weights.reshape(B, S, 1)

    slotg3, mask3, idx3 = pl.pallas_call(
        _slot_body,
        out_shape=[
            jax.ShapeDtypeStruct((B, 1, S), jnp.int32),
            jax.ShapeDtypeStruct((B, 1, S), jnp.float32),
            jax.ShapeDtypeStruct((B, C, 1), jnp.int32),
        ],
        grid=(B,),
        in_specs=[
            pl.BlockSpec((1, 1, S), lambda b: (b, 0, 0)),
            pl.BlockSpec((1, S, 1), lambda b: (b, 0, 0)),
        ],
        out_specs=[
            pl.BlockSpec((1, 1, S), lambda b: (b, 0, 0)),
            pl.BlockSpec((1, 1, S), lambda b: (b, 0, 0)),
            pl.BlockSpec((1, C, 1), lambda b: (b, 0, 0)),
        ],
    )(w_row3, w_col3)
    slotg = slotg3.reshape(B * S)
    mask = mask3.reshape(B * S, 1)
    idxg = idx3.reshape(BC)

    mesh = plsc.VectorSubcoreMesh(core_axis_name="c", subcore_axis_name="s")
    xg = pl.kernel(
        _sc_compact_body,
        out_type=jax.ShapeDtypeStruct((BC, D), jnp.float32),
        mesh=mesh,
        scratch_types=[
            pltpu.VMEM((PER_TILE,), jnp.int32),
            pltpu.VMEM((128, D), jnp.float32),
            pltpu.SemaphoreType.DMA,
        ],
    )(idxg, hid)

    wv_b = wv.astype(jnp.bfloat16)
    wo_b = wo.astype(jnp.bfloat16)
    wg_b = wg.astype(jnp.bfloat16)
    wu_b = wu.astype(jnp.bfloat16)
    wd_b = wd.astype(jnp.bfloat16)
    g1r = g1.reshape(1, D)
    g2r = g2.reshape(1, D)

    yg = pl.pallas_call(
        _block_body,
        out_shape=jax.ShapeDtypeStruct((BC, D), jnp.float32),
        grid=(BC // TOK_BLK,),
        in_specs=[
            pl.BlockSpec((TOK_BLK, D), lambda i: (i, 0)),
            pl.BlockSpec((D, D), lambda i: (0, 0)),
            pl.BlockSpec((D, D), lambda i: (0, 0)),
            pl.BlockSpec((1, D), lambda i: (0, 0)),
            pl.BlockSpec((1, D), lambda i: (0, 0)),
            pl.BlockSpec((D, DFF), lambda i: (0, 0)),
            pl.BlockSpec((D, DFF), lambda i: (0, 0)),
            pl.BlockSpec((DFF, D), lambda i: (0, 0)),
        ],
        out_specs=pl.BlockSpec((TOK_BLK, D), lambda i: (i, 0)),
        scratch_shapes=[pltpu.VMEM((D, D), jnp.bfloat16)],
        compiler_params=pltpu.CompilerParams(
            dimension_semantics=("arbitrary",)),
    )(xg, wv_b, wo_b, g1r, g2r, wg_b, wu_b, wd_b)

    zg = pl.kernel(
        _sc_expand_body,
        out_type=jax.ShapeDtypeStruct((B * S, D), jnp.float32),
        mesh=mesh,
        scratch_types=[
            pltpu.VMEM((EXP_TILE,), jnp.int32),
            pltpu.VMEM((128, D), jnp.float32),
            pltpu.SemaphoreType.DMA,
        ],
    )(slotg, yg)

    out = pl.pallas_call(
        _blend_body,
        out_shape=jax.ShapeDtypeStruct((B * S, D), jnp.float32),
        grid=(B * S // ROUTER_BLK,),
        in_specs=[
            pl.BlockSpec((ROUTER_BLK, D), lambda i: (i, 0)),
            pl.BlockSpec((ROUTER_BLK, D), lambda i: (i, 0)),
            pl.BlockSpec((ROUTER_BLK, 1), lambda i: (i, 0)),
        ],
        out_specs=pl.BlockSpec((ROUTER_BLK, D), lambda i: (i, 0)),
    )(hid, zg, mask)

    return out.reshape(B, S, D)
